# baseline (speedup 1.0000x reference)
import functools

import jax
import jax.numpy as jnp
from jax import lax
from jax.experimental import pallas as pl
from jax.experimental.pallas import tpu as pltpu
from jax.experimental.pallas import tpu_sc as plsc

H = 32
W = 32
D = 128
L = 16
NCORES = 1
NSUB = 16
NWORK = NCORES * NSUB
HPW = H // NWORK
RPW = HPW * W


def _pos_embed_body(row_hbm, col_hbm, out_hbm, idxbuf, outbuf,
                    sem_r, sem_o, *sem_cs):
    wid = lax.axis_index("s") * NCORES + lax.axis_index("c")
    cp_cols = [
        pltpu.make_async_copy(
            col_hbm, outbuf.at[pl.ds(j * W, W), pl.ds(0, D)], sem_cs[j])
        for j in range(HPW)
    ]
    for cp in cp_cols:
        cp.start()
    for j in range(HPW):
        hval = jnp.full((L,), wid * HPW + j, jnp.int32)
        for c in range(W // L):
            idxbuf[pl.ds(j * W + c * L, L)] = hval
    cp_r = pltpu.make_async_copy(
        row_hbm.at[idxbuf], outbuf.at[:, pl.ds(D, D)], sem_r)
    cp_r.start()
    cp_r.wait()
    for cp in cp_cols:
        cp.wait()
    pltpu.sync_copy(outbuf, out_hbm.at[pl.ds(wid * RPW, RPW)])


@jax.jit
def _pos_embed(row_embed, col_embed):
    mesh = plsc.VectorSubcoreMesh(
        core_axis_name="c", subcore_axis_name="s",
        num_cores=NCORES, num_subcores=NSUB)
    kfn = functools.partial(
        pl.kernel,
        mesh=mesh,
        out_type=jax.ShapeDtypeStruct((H * W, 2 * D), jnp.float32),
        scratch_types=[
            pltpu.VMEM((RPW,), jnp.int32),
            pltpu.VMEM((RPW, 2 * D), jnp.float32),
            pltpu.SemaphoreType.DMA,
            pltpu.SemaphoreType.DMA,
        ] + [pltpu.SemaphoreType.DMA for _ in range(HPW)],
    )(_pos_embed_body)
    return kfn(row_embed, col_embed)


def kernel(tensor, row_embed, col_embed):
    del tensor
    return _pos_embed(row_embed, col_embed)


# speedup vs baseline: 1.0782x; 1.0782x over previous
import functools

import jax
import jax.numpy as jnp
from jax import lax
from jax.experimental import pallas as pl
from jax.experimental.pallas import tpu as pltpu
from jax.experimental.pallas import tpu_sc as plsc

H = 32
W = 32
D = 128
L = 16
NCORES = 1
NSUB = 16
NWORK = NCORES * NSUB
HPW = H // NWORK
RPW = HPW * W


def _pos_embed_body(row_hbm, col_hbm, out_hbm, rowbuf, outbuf,
                    sem_r, sem_o, *sem_cs):
    wid = lax.axis_index("s") * NCORES + lax.axis_index("c")
    cp_cols = [
        pltpu.make_async_copy(
            col_hbm, outbuf.at[pl.ds(j * W, W), pl.ds(0, D)], sem_cs[j])
        for j in range(HPW)
    ]
    cp_r = pltpu.make_async_copy(
        row_hbm.at[pl.ds(wid * HPW, HPW)], rowbuf, sem_r)
    for cp in cp_cols:
        cp.start()
    cp_r.start()
    cp_r.wait()
    cp_outs = [
        pltpu.make_async_copy(
            outbuf.at[pl.ds(j * W, W)],
            out_hbm.at[pl.ds((wid * HPW + j) * W, W)], sem_o)
        for j in range(HPW)
    ]
    for j in range(HPW):
        rvecs = [rowbuf[j, pl.ds(k * L, L)] for k in range(D // L)]

        def bcast(w, carry, _j=j, _rvecs=rvecs):
            for k in range(D // L):
                outbuf[_j * W + w, pl.ds(D + k * L, L)] = _rvecs[k]
            return carry

        lax.fori_loop(0, W, bcast, 0, unroll=False)
        cp_cols[j].wait()
        cp_outs[j].start()
    for cp in cp_outs:
        cp.wait()


@jax.jit
def _pos_embed(row_embed, col_embed):
    mesh = plsc.VectorSubcoreMesh(
        core_axis_name="c", subcore_axis_name="s",
        num_cores=NCORES, num_subcores=NSUB)
    kfn = functools.partial(
        pl.kernel,
        mesh=mesh,
        out_type=jax.ShapeDtypeStruct((H * W, 2 * D), jnp.float32),
        scratch_types=[
            pltpu.VMEM((HPW, D), jnp.float32),
            pltpu.VMEM((RPW, 2 * D), jnp.float32),
            pltpu.SemaphoreType.DMA,
            pltpu.SemaphoreType.DMA,
        ] + [pltpu.SemaphoreType.DMA for _ in range(HPW)],
    )(_pos_embed_body)
    return kfn(row_embed, col_embed)


def kernel(tensor, row_embed, col_embed):
    del tensor
    return _pos_embed(row_embed, col_embed)


# speedup vs baseline: 1.0995x; 1.0198x over previous
import functools

import jax
import jax.numpy as jnp
from jax import lax
from jax.experimental import pallas as pl
from jax.experimental.pallas import tpu as pltpu
from jax.experimental.pallas import tpu_sc as plsc

H = 32
W = 32
D = 128
L = 16
NCORES = 1
NSUB = 16
NWORK = NCORES * NSUB
HPW = H // NWORK
RPW = HPW * W


def _pos_embed_body(row_hbm, col_hbm, out_hbm, rowbuf, outbuf,
                    sem_r, sem_o, *sem_cs):
    wid = lax.axis_index("s") * NCORES + lax.axis_index("c")
    cp_cols = [
        pltpu.make_async_copy(
            col_hbm, outbuf.at[pl.ds(j * W, W), pl.ds(0, D)], sem_cs[j])
        for j in range(HPW)
    ]
    cp_r = pltpu.make_async_copy(
        row_hbm.at[pl.ds(wid * HPW, HPW)], rowbuf, sem_r)
    for cp in cp_cols:
        cp.start()
    cp_r.start()
    cp_r.wait()
    rvecs = [[rowbuf[j, pl.ds(k * L, L)] for k in range(D // L)]
             for j in range(HPW)]

    def bcast(w, carry):
        for j in range(HPW):
            for k in range(D // L):
                outbuf[j * W + w, pl.ds(D + k * L, L)] = rvecs[j][k]
        return carry

    lax.fori_loop(0, W, bcast, 0, unroll=2)
    for cp in cp_cols:
        cp.wait()
    pltpu.sync_copy(outbuf, out_hbm.at[pl.ds(wid * RPW, RPW)])


@jax.jit
def _pos_embed(row_embed, col_embed):
    mesh = plsc.VectorSubcoreMesh(
        core_axis_name="c", subcore_axis_name="s",
        num_cores=NCORES, num_subcores=NSUB)
    kfn = functools.partial(
        pl.kernel,
        mesh=mesh,
        out_type=jax.ShapeDtypeStruct((H * W, 2 * D), jnp.float32),
        scratch_types=[
            pltpu.VMEM((HPW, D), jnp.float32),
            pltpu.VMEM((RPW, 2 * D), jnp.float32),
            pltpu.SemaphoreType.DMA,
            pltpu.SemaphoreType.DMA,
        ] + [pltpu.SemaphoreType.DMA for _ in range(HPW)],
    )(_pos_embed_body)
    return kfn(row_embed, col_embed)


def kernel(tensor, row_embed, col_embed):
    del tensor
    return _pos_embed(row_embed, col_embed)


# speedup vs baseline: 1.1047x; 1.0047x over previous
import functools

import jax
import jax.numpy as jnp
from jax import lax
from jax.experimental import pallas as pl
from jax.experimental.pallas import tpu as pltpu
from jax.experimental.pallas import tpu_sc as plsc

H = 32
W = 32
D = 128
L = 16
NCORES = 1
NSUB = 16
NWORK = NCORES * NSUB
HPW = H // NWORK
RPW = HPW * W


def _pos_embed_body(row_hbm, col_hbm, out_hbm, rowbuf, outbuf,
                    sem_r, sem_o, *sem_cs):
    wid = lax.axis_index("s") * NCORES + lax.axis_index("c")
    cp_cols = [
        pltpu.make_async_copy(
            col_hbm, outbuf.at[pl.ds(j * W, W), pl.ds(0, D)], sem_cs[j])
        for j in range(HPW)
    ]
    cp_r = pltpu.make_async_copy(
        row_hbm.at[pl.ds(wid * HPW, HPW)], rowbuf, sem_r)
    for cp in cp_cols:
        cp.start()
    cp_r.start()
    cp_r.wait()
    rvecs = [[rowbuf[j, pl.ds(k * L, L)] for k in range(D // L)]
             for j in range(HPW)]

    @plsc.parallel_loop(0, W, step=1)
    def bcast(w):
        for j in range(HPW):
            for k in range(D // L):
                outbuf[j * W + w, pl.ds(D + k * L, L)] = rvecs[j][k]
    for cp in cp_cols:
        cp.wait()
    pltpu.sync_copy(outbuf, out_hbm.at[pl.ds(wid * RPW, RPW)])


@jax.jit
def _pos_embed(row_embed, col_embed):
    mesh = plsc.VectorSubcoreMesh(
        core_axis_name="c", subcore_axis_name="s",
        num_cores=NCORES, num_subcores=NSUB)
    kfn = functools.partial(
        pl.kernel,
        mesh=mesh,
        out_type=jax.ShapeDtypeStruct((H * W, 2 * D), jnp.float32),
        scratch_types=[
            pltpu.VMEM((HPW, D), jnp.float32),
            pltpu.VMEM((RPW, 2 * D), jnp.float32),
            pltpu.SemaphoreType.DMA,
            pltpu.SemaphoreType.DMA,
        ] + [pltpu.SemaphoreType.DMA for _ in range(HPW)],
    )(_pos_embed_body)
    return kfn(row_embed, col_embed)


def kernel(tensor, row_embed, col_embed):
    del tensor
    return _pos_embed(row_embed, col_embed)
